# Initial kernel scaffold; baseline (speedup 1.0000x reference)
import jax
import jax.numpy as jnp
from jax.experimental import pallas as pl


def kernel(X, S, mask, params):
    raise NotImplementedError("write your pallas kernel here")


# speedup vs baseline: 5.9210x; 5.9210x over previous
import functools

import jax
import jax.numpy as jnp
import numpy as np
from jax import lax
from jax.experimental import pallas as pl
from jax.experimental.pallas import tpu as pltpu

B, L, H, K, VOCAB = 4, 512, 256, 16, 4
N_ENC, N_DEC = 3, 3
N_RBF, N_POS = 16, 16
NODE_IN = 9
EDGE_IN = N_RBF + N_POS

N = B * L
E = B * L * K
EBLK = 4096
NBLK = EBLK // K
N_EBLKS = E // EBLK
BLKS_PER_BATCH = (L * K) // EBLK


def _ln(x, g, b):
    mu = jnp.mean(x, -1, keepdims=True)
    var = jnp.var(x, -1, keepdims=True)
    return (x - mu) / jnp.sqrt(var + 1e-5) * g + b


def _prep_body(nraw_ref, soh_ref, mask_ref, nw_ref, nb_ref, ws_ref,
               g_ref, b_ref, w1s_ref, w1d_ref,
               hv_ref, a_ref, bd_ref):
    h = jnp.dot(nraw_ref[...], nw_ref[...], preferred_element_type=jnp.float32)
    h = h + nb_ref[...] + jnp.dot(soh_ref[...], ws_ref[...],
                                  preferred_element_type=jnp.float32)
    hv = _ln(h, g_ref[...], b_ref[...]) * mask_ref[...]
    hv_ref[...] = hv
    a_ref[...] = jnp.dot(hv, w1s_ref[...], preferred_element_type=jnp.float32)
    bd_ref[...] = jnp.dot(hv, w1d_ref[...], preferred_element_type=jnp.float32)


def _eemb_body(eraw_ref, ew_ref, eb_ref, g_ref, b_ref, he_ref):
    h = jnp.dot(eraw_ref[...], ew_ref[...], preferred_element_type=jnp.float32)
    he_ref[...] = _ln(h + eb_ref[...], g_ref[...], b_ref[...])


def _edge_body(src_ref, he_ref, a_ref, bd_ref, w1e_ref, b1_ref,
               w2_ref, b2_ref, agg_ref):
    pid = pl.program_id(0)
    base = (pid // BLKS_PER_BATCH) * L
    src_local = src_ref[0, 0, :] - base
    onehot = (src_local[:, None] ==
              lax.broadcasted_iota(jnp.int32, (EBLK, L), 1)
              ).astype(jnp.float32)
    m = jnp.dot(he_ref[...], w1e_ref[...],
                preferred_element_type=jnp.float32)
    m = m + b1_ref[...] + jnp.dot(onehot, a_ref[...],
                                  preferred_element_type=jnp.float32)
    m3 = m.reshape(NBLK, K, H) + bd_ref[...][:, None, :]
    m = jax.nn.gelu(m3).reshape(EBLK, H)
    m = jax.nn.gelu(jnp.dot(m, w2_ref[...],
                            preferred_element_type=jnp.float32) + b2_ref[...])
    agg_ref[...] = jnp.sum(m.reshape(NBLK, K, H), axis=1)


def _node_body(hv_ref, agg_ref, w3_ref, b3_ref, g1_ref, bb1_ref,
               f1_ref, fb1_ref, f2_ref, fb2_ref, g2_ref, bb2_ref,
               wsk_ref, bsk_ref, gsk_ref, bsk2_ref,
               w1s_ref, w1d_ref,
               hv_out, a_out, bd_out):
    hv = hv_ref[...]
    agg = jnp.dot(agg_ref[...] * (1.0 / K), w3_ref[...],
                  preferred_element_type=jnp.float32) + b3_ref[...]
    h = _ln(hv + agg, g1_ref[...], bb1_ref[...])
    ff = jnp.dot(jnp.maximum(
        jnp.dot(h, f1_ref[...], preferred_element_type=jnp.float32)
        + fb1_ref[...], 0.0), f2_ref[...],
        preferred_element_type=jnp.float32) + fb2_ref[...]
    h = _ln(h + ff, g2_ref[...], bb2_ref[...])
    sk = jnp.maximum(jnp.dot(h, wsk_ref[...],
                             preferred_element_type=jnp.float32)
                     + bsk_ref[...], 0.0)
    hv_new = hv + _ln(sk, gsk_ref[...], bsk2_ref[...])
    hv_out[...] = hv_new
    a_out[...] = jnp.dot(hv_new, w1s_ref[...],
                         preferred_element_type=jnp.float32)
    bd_out[...] = jnp.dot(hv_new, w1d_ref[...],
                          preferred_element_type=jnp.float32)


def _readout_body(hv_ref, wr_ref, br_ref, p1_ref, p2_ref, p2b_ref,
                  logits_ref, prj_ref):
    hv = hv_ref[...]
    logits_ref[...] = jnp.dot(hv, wr_ref[...],
                              preferred_element_type=jnp.float32) + br_ref[...]
    ge = jnp.mean(hv.reshape(2, L, H), axis=1)
    prj = jnp.maximum(jnp.dot(ge, p1_ref[...],
                              preferred_element_type=jnp.float32), 0.0)
    prj = jnp.dot(prj, p2_ref[...],
                  preferred_element_type=jnp.float32) + p2b_ref[...]
    prj_ref[...] = prj.reshape(1, 2, H)


def _full(full_shape):
    return pl.BlockSpec(full_shape, lambda *_: tuple(0 for _ in full_shape))


def kernel(X, S, mask, params):
    p = params
    f32 = jnp.float32

    center = X[:, :, 1, :]
    diff = center[:, :, None, :] - center[:, None, :, :]
    D = jnp.sqrt(jnp.sum(diff * diff, -1) + 1e-8)
    D = D + jnp.eye(L, dtype=f32)[None] * 1e6
    negD, nbr = jax.lax.top_k(-D, K)
    d_nbr = -negD
    centers = jnp.linspace(2.0, 22.0, N_RBF)
    sigma = (22.0 - 2.0) / N_RBF
    rbf = jnp.exp(-(((d_nbr[..., None] - centers) / sigma) ** 2))
    rel = (nbr - jnp.arange(L)[None, :, None]).astype(f32)
    freq = jnp.exp(-jnp.arange(N_POS // 2, dtype=f32)
                   * (np.log(10000.0) / (N_POS // 2)))
    ang = rel[..., None] * freq
    posenc = jnp.concatenate([jnp.sin(ang), jnp.cos(ang)], -1)
    e_raw = jnp.concatenate([rbf, posenc], -1).reshape(E, EDGE_IN)

    def unit(v):
        return v / (jnp.linalg.norm(v, axis=-1, keepdims=True) + 1e-8)
    v1 = unit(X[:, :, 1] - X[:, :, 0])
    v2 = unit(X[:, :, 2] - X[:, :, 1])
    v3 = unit(jnp.roll(center, -1, axis=1) - center)
    node_raw = jnp.concatenate([v1, v2, v3], -1).reshape(N, NODE_IN)

    offs = (jnp.arange(B, dtype=jnp.int32) * L)[:, None, None]
    src = (nbr.astype(jnp.int32) + offs).reshape(N_EBLKS, 1, EBLK)
    s_oh = jax.nn.one_hot(S.reshape(N), VOCAB, dtype=f32)
    mask_col = mask.reshape(N, 1)

    row = lambda v: v.reshape(1, -1)

    hv, a, bd = pl.pallas_call(
        _prep_body,
        grid=(),
        in_specs=[_full((N, NODE_IN)), _full((N, VOCAB)), _full((N, 1)),
                  _full((NODE_IN, H)), _full((1, H)), _full((VOCAB, H)),
                  _full((1, H)), _full((1, H)), _full((H, H)), _full((H, H))],
        out_specs=[_full((N, H))] * 3,
        out_shape=[jax.ShapeDtypeStruct((N, H), f32)] * 3,
    )(node_raw, s_oh, mask_col, p['node_W'], row(p['node_b']), p['W_s'],
      row(p['node_ln_g']), row(p['node_ln_b']),
      p['enc0_m1_W'][H:2 * H], p['enc0_m1_W'][2 * H:])

    he = pl.pallas_call(
        _eemb_body,
        grid=(N_EBLKS,),
        in_specs=[pl.BlockSpec((EBLK, EDGE_IN), lambda i: (i, 0)),
                  _full((EDGE_IN, H)), _full((1, H)),
                  _full((1, H)), _full((1, H))],
        out_specs=pl.BlockSpec((EBLK, H), lambda i: (i, 0)),
        out_shape=jax.ShapeDtypeStruct((E, H), f32),
        compiler_params=pltpu.CompilerParams(
            dimension_semantics=("parallel",)),
    )(e_raw, p['edge_W'], row(p['edge_b']),
      row(p['edge_ln_g']), row(p['edge_ln_b']))

    layers = ['enc%d' % i for i in range(N_ENC)] + \
             ['dec%d' % i for i in range(N_DEC)]
    for li, pre in enumerate(layers):
        agg = pl.pallas_call(
            _edge_body,
            grid=(N_EBLKS,),
            in_specs=[pl.BlockSpec((1, 1, EBLK), lambda i: (i, 0, 0)),
                      pl.BlockSpec((EBLK, H), lambda i: (i, 0)),
                      pl.BlockSpec((L, H), lambda i: (i // BLKS_PER_BATCH, 0)),
                      pl.BlockSpec((NBLK, H), lambda i: (i, 0)),
                      _full((H, H)), _full((1, H)),
                      _full((H, H)), _full((1, H))],
            out_specs=pl.BlockSpec((NBLK, H), lambda i: (i, 0)),
            out_shape=jax.ShapeDtypeStruct((N, H), f32),
            compiler_params=pltpu.CompilerParams(
                dimension_semantics=("parallel",)),
        )(src, he, a, bd,
          p[pre + '_m1_W'][:H], row(p[pre + '_m1_b']),
          p[pre + '_m2_W'], row(p[pre + '_m2_b']))

        nxt = layers[li + 1] if li + 1 < len(layers) else None
        w1s_next = p[nxt + '_m1_W'][H:2 * H] if nxt else jnp.zeros((H, H), f32)
        w1d_next = p[nxt + '_m1_W'][2 * H:] if nxt else jnp.zeros((H, H), f32)
        hv, a, bd = pl.pallas_call(
            _node_body,
            grid=(2,),
            in_specs=[pl.BlockSpec((N // 2, H), lambda i: (i, 0)),
                      pl.BlockSpec((N // 2, H), lambda i: (i, 0)),
                      _full((H, H)), _full((1, H)),
                      _full((1, H)), _full((1, H)),
                      _full((H, 4 * H)), _full((1, 4 * H)),
                      _full((4 * H, H)), _full((1, H)),
                      _full((1, H)), _full((1, H)),
                      _full((H, H)), _full((1, H)),
                      _full((1, H)), _full((1, H)),
                      _full((H, H)), _full((H, H))],
            out_specs=[pl.BlockSpec((N // 2, H), lambda i: (i, 0))] * 3,
            out_shape=[jax.ShapeDtypeStruct((N, H), f32)] * 3,
            compiler_params=pltpu.CompilerParams(
                dimension_semantics=("parallel",)),
        )(hv, agg,
          p[pre + '_m3_W'], row(p[pre + '_m3_b']),
          row(p[pre + '_ln1_g']), row(p[pre + '_ln1_b']),
          p[pre + '_f1_W'], row(p[pre + '_f1_b']),
          p[pre + '_f2_W'], row(p[pre + '_f2_b']),
          row(p[pre + '_ln2_g']), row(p[pre + '_ln2_b']),
          p[pre + '_skip_W'], row(p[pre + '_skip_b']),
          row(p[pre + '_skln_g']), row(p[pre + '_skln_b']),
          w1s_next, w1d_next)

    logits, prjs = pl.pallas_call(
        _readout_body,
        grid=(2,),
        in_specs=[pl.BlockSpec((N // 2, H), lambda i: (i, 0)),
                  _full((H, VOCAB)), _full((1, VOCAB)),
                  _full((H, H)), _full((H, H)), _full((1, H))],
        out_specs=[pl.BlockSpec((N // 2, VOCAB), lambda i: (i, 0)),
                   pl.BlockSpec((1, 2, H), lambda i: (i, 0, 0))],
        out_shape=[jax.ShapeDtypeStruct((N, VOCAB), f32),
                   jax.ShapeDtypeStruct((2, 2, H), f32)],
        compiler_params=pltpu.CompilerParams(
            dimension_semantics=("parallel",)),
    )(hv, p['readout_W'], row(p['readout_b']),
      p['proj1_W'], p['proj2_W'], row(p['proj2_b']))

    return logits, S.reshape(-1), prjs.reshape(B, H)


# speedup vs baseline: 5.9707x; 1.0084x over previous
import functools

import jax
import jax.numpy as jnp
import numpy as np
from jax import lax
from jax.experimental import pallas as pl
from jax.experimental.pallas import tpu as pltpu

B, L, H, K, VOCAB = 4, 512, 256, 16, 4
N_ENC, N_DEC = 3, 3
N_RBF, N_POS = 16, 16
NODE_IN = 9
EDGE_IN = N_RBF + N_POS

N = B * L
E = B * L * K
NLAYERS = N_ENC + N_DEC
EBLK = 2048
NBLK = EBLK // K
N_EBLKS = E // EBLK
BLKS_PER_BATCH = (L * K) // EBLK
f32 = jnp.float32
bf16 = jnp.bfloat16


def _ln(x, g, b):
    mu = jnp.mean(x, -1, keepdims=True)
    var = jnp.var(x, -1, keepdims=True)
    return (x - mu) / jnp.sqrt(var + 1e-5) * g + b


def _eemb_body(eraw_ref, ew_ref, eb_ref, g_ref, b_ref, he_ref):
    h = jnp.dot(eraw_ref[...], ew_ref[...], preferred_element_type=f32)
    he_ref[...] = _ln(h + eb_ref[...], g_ref[...], b_ref[...]).astype(bf16)


def _mega_body(nraw_ref, soh_ref, mask_ref, src_ref, he_hbm,
               nw_ref, nb_ref, ws_ref, nlg_ref, nlb_ref,
               w1e_ref, w1s_ref, w1d_ref, b1_ref,
               w2_ref, b2_ref, w3_ref, b3_ref,
               g1_ref, bb1_ref, f1_ref, fb1_ref, f2_ref, fb2_ref,
               g2_ref, bb2_ref, wsk_ref, bsk_ref, gsk_ref, bsk2_ref,
               wr_ref, br_ref, p1_ref, p2_ref, p2b_ref,
               logits_ref, prj_ref,
               hv_s, a_s, bd_s, agg_s, he_buf, sem):
    h = jnp.dot(nraw_ref[...], nw_ref[...], preferred_element_type=f32)
    h = h + nb_ref[...] + jnp.dot(soh_ref[...], ws_ref[...],
                                  preferred_element_type=f32)
    hv0 = _ln(h, nlg_ref[...], nlb_ref[...]) * mask_ref[...]
    hv_s[...] = hv0
    a_s[...] = jnp.dot(hv0, w1s_ref[0], preferred_element_type=f32)
    bd_s[...] = jnp.dot(hv0, w1d_ref[0], preferred_element_type=f32)

    def he_copy(blk, slot):
        return pltpu.make_async_copy(
            he_hbm.at[pl.ds(blk * EBLK, EBLK), :], he_buf.at[slot],
            sem.at[slot])

    def layer_body(l, carry):
        w1e_b = w1e_ref[l].astype(bf16)
        b1v = b1_ref[l]
        w2v = w2_ref[l]
        b2v = b2_ref[l]

        he_copy(0, 0).start()

        def edge_body(blk, c):
            slot = lax.rem(blk, 2)
            he_copy(blk, slot).wait()

            @pl.when(blk + 1 < N_EBLKS)
            def _():
                he_copy(blk + 1, lax.rem(blk + 1, 2)).start()

            base = (blk // BLKS_PER_BATCH) * L
            src_local = (src_ref[blk, 0, :] - base).astype(jnp.int16)
            oh = jnp.where(
                src_local[:, None] ==
                lax.broadcasted_iota(jnp.int16, (EBLK, L), 1),
                bf16(1.0), bf16(0.0))
            a_blk = a_s[pl.ds(base, L), :].astype(bf16)
            gath = jnp.dot(oh, a_blk, preferred_element_type=f32)
            epart = jnp.dot(he_buf[slot], w1e_b,
                            preferred_element_type=f32)
            pre = (epart + b1v + gath).reshape(NBLK, K, H) \
                + bd_s[pl.ds(blk * NBLK, NBLK), :][:, None, :]
            m = jax.nn.gelu(pre).reshape(EBLK, H)
            m = jax.nn.gelu(jnp.dot(m, w2v, preferred_element_type=f32)
                            + b2v)
            agg_s[pl.ds(blk * NBLK, NBLK), :] = \
                jnp.sum(m.reshape(NBLK, K, H), axis=1)
            return c

        lax.fori_loop(0, N_EBLKS, edge_body, 0)

        hv = hv_s[...]
        agg = jnp.dot(agg_s[...] * (1.0 / K), w3_ref[l],
                      preferred_element_type=f32) + b3_ref[l]
        hh = _ln(hv + agg, g1_ref[l], bb1_ref[l])
        ff = jnp.dot(jnp.maximum(
            jnp.dot(hh, f1_ref[l], preferred_element_type=f32)
            + fb1_ref[l], 0.0), f2_ref[l],
            preferred_element_type=f32) + fb2_ref[l]
        hh = _ln(hh + ff, g2_ref[l], bb2_ref[l])
        sk = jnp.maximum(jnp.dot(hh, wsk_ref[l],
                                 preferred_element_type=f32)
                         + bsk_ref[l], 0.0)
        hv_new = hv + _ln(sk, gsk_ref[l], bsk2_ref[l])
        hv_s[...] = hv_new
        ln = jnp.minimum(l + 1, NLAYERS - 1)
        a_s[...] = jnp.dot(hv_new, w1s_ref[ln], preferred_element_type=f32)
        bd_s[...] = jnp.dot(hv_new, w1d_ref[ln], preferred_element_type=f32)
        return carry

    lax.fori_loop(0, NLAYERS, layer_body, 0)

    hv = hv_s[...]
    logits_ref[...] = jnp.dot(hv, wr_ref[...],
                              preferred_element_type=f32) + br_ref[...]
    ge = jnp.mean(hv.reshape(B, L, H), axis=1)
    prj = jnp.maximum(jnp.dot(ge, p1_ref[...],
                              preferred_element_type=f32), 0.0)
    prj_ref[...] = jnp.dot(prj, p2_ref[...],
                           preferred_element_type=f32) + p2b_ref[...]


def _full(shape):
    return pl.BlockSpec(shape, lambda *_: tuple(0 for _ in shape))


def kernel(X, S, mask, params):
    p = params

    center = X[:, :, 1, :]
    diff = center[:, :, None, :] - center[:, None, :, :]
    D = jnp.sqrt(jnp.sum(diff * diff, -1) + 1e-8)
    D = D + jnp.eye(L, dtype=f32)[None] * 1e6
    negD, nbr = jax.lax.top_k(-D, K)
    d_nbr = -negD
    centers = jnp.linspace(2.0, 22.0, N_RBF)
    sigma = (22.0 - 2.0) / N_RBF
    rbf = jnp.exp(-(((d_nbr[..., None] - centers) / sigma) ** 2))
    rel = (nbr - jnp.arange(L)[None, :, None]).astype(f32)
    freq = jnp.exp(-jnp.arange(N_POS // 2, dtype=f32)
                   * (np.log(10000.0) / (N_POS // 2)))
    ang = rel[..., None] * freq
    posenc = jnp.concatenate([jnp.sin(ang), jnp.cos(ang)], -1)
    e_raw = jnp.concatenate([rbf, posenc], -1).reshape(E, EDGE_IN)

    def unit(v):
        return v / (jnp.linalg.norm(v, axis=-1, keepdims=True) + 1e-8)
    v1 = unit(X[:, :, 1] - X[:, :, 0])
    v2 = unit(X[:, :, 2] - X[:, :, 1])
    v3 = unit(jnp.roll(center, -1, axis=1) - center)
    node_raw = jnp.concatenate([v1, v2, v3], -1).reshape(N, NODE_IN)

    offs = (jnp.arange(B, dtype=jnp.int32) * L)[:, None, None]
    src = (nbr.astype(jnp.int32) + offs).reshape(N_EBLKS, 1, EBLK)
    s_oh = jax.nn.one_hot(S.reshape(N), VOCAB, dtype=f32)
    mask_col = mask.reshape(N, 1)

    row = lambda v: v.reshape(1, -1)
    layers = ['enc%d' % i for i in range(N_ENC)] + \
             ['dec%d' % i for i in range(N_DEC)]
    stk = lambda name: jnp.stack([p[pre + name] for pre in layers])
    stkr = lambda name: jnp.stack([row(p[pre + name]) for pre in layers])

    he = pl.pallas_call(
        _eemb_body,
        grid=(8,),
        in_specs=[pl.BlockSpec((E // 8, EDGE_IN), lambda i: (i, 0)),
                  _full((EDGE_IN, H)), _full((1, H)),
                  _full((1, H)), _full((1, H))],
        out_specs=pl.BlockSpec((E // 8, H), lambda i: (i, 0)),
        out_shape=jax.ShapeDtypeStruct((E, H), bf16),
        compiler_params=pltpu.CompilerParams(
            dimension_semantics=("parallel",)),
    )(e_raw, p['edge_W'], row(p['edge_b']),
      row(p['edge_ln_g']), row(p['edge_ln_b']))

    logits, prjs = pl.pallas_call(
        _mega_body,
        grid=(),
        in_specs=[_full((N, NODE_IN)), _full((N, VOCAB)), _full((N, 1)),
                  _full((N_EBLKS, 1, EBLK)),
                  pl.BlockSpec(memory_space=pl.ANY),
                  _full((NODE_IN, H)), _full((1, H)), _full((VOCAB, H)),
                  _full((1, H)), _full((1, H)),
                  _full((NLAYERS, H, H)), _full((NLAYERS, H, H)),
                  _full((NLAYERS, H, H)), _full((NLAYERS, 1, H)),
                  _full((NLAYERS, H, H)), _full((NLAYERS, 1, H)),
                  _full((NLAYERS, H, H)), _full((NLAYERS, 1, H)),
                  _full((NLAYERS, 1, H)), _full((NLAYERS, 1, H)),
                  _full((NLAYERS, H, 4 * H)), _full((NLAYERS, 1, 4 * H)),
                  _full((NLAYERS, 4 * H, H)), _full((NLAYERS, 1, H)),
                  _full((NLAYERS, 1, H)), _full((NLAYERS, 1, H)),
                  _full((NLAYERS, H, H)), _full((NLAYERS, 1, H)),
                  _full((NLAYERS, 1, H)), _full((NLAYERS, 1, H)),
                  _full((H, VOCAB)), _full((1, VOCAB)),
                  _full((H, H)), _full((H, H)), _full((1, H))],
        out_specs=[_full((N, VOCAB)), _full((B, H))],
        out_shape=[jax.ShapeDtypeStruct((N, VOCAB), f32),
                   jax.ShapeDtypeStruct((B, H), f32)],
        scratch_shapes=[pltpu.VMEM((N, H), f32), pltpu.VMEM((N, H), f32),
                        pltpu.VMEM((N, H), f32), pltpu.VMEM((N, H), f32),
                        pltpu.VMEM((2, EBLK, H), bf16),
                        pltpu.SemaphoreType.DMA((2,))],
    )(node_raw, s_oh, mask_col, src, he,
      p['node_W'], row(p['node_b']), p['W_s'],
      row(p['node_ln_g']), row(p['node_ln_b']),
      stk('_m1_W')[:, :H], stk('_m1_W')[:, H:2 * H], stk('_m1_W')[:, 2 * H:],
      stkr('_m1_b'),
      stk('_m2_W'), stkr('_m2_b'), stk('_m3_W'), stkr('_m3_b'),
      stkr('_ln1_g'), stkr('_ln1_b'),
      stk('_f1_W'), stkr('_f1_b'), stk('_f2_W'), stkr('_f2_b'),
      stkr('_ln2_g'), stkr('_ln2_b'),
      stk('_skip_W'), stkr('_skip_b'), stkr('_skln_g'), stkr('_skln_b'),
      p['readout_W'], row(p['readout_b']),
      p['proj1_W'], p['proj2_W'], row(p['proj2_b']))

    return logits, S.reshape(-1), prjs


# speedup vs baseline: 6.4158x; 1.0745x over previous
import functools

import jax
import jax.numpy as jnp
import numpy as np
from jax import lax
from jax.experimental import pallas as pl
from jax.experimental.pallas import tpu as pltpu

B, L, H, K, VOCAB = 4, 512, 256, 16, 4
N_ENC, N_DEC = 3, 3
N_RBF, N_POS = 16, 16
NODE_IN = 9
EDGE_IN = N_RBF + N_POS

N = B * L
E = B * L * K
NLAYERS = N_ENC + N_DEC
EBLK = 2048
NBLK = EBLK // K
N_EBLKS = E // EBLK
BLKS_PER_BATCH = (L * K) // EBLK
f32 = jnp.float32
bf16 = jnp.bfloat16


def _ln(x, g, b):
    mu = jnp.mean(x, -1, keepdims=True)
    var = jnp.var(x, -1, keepdims=True)
    return (x - mu) / jnp.sqrt(var + 1e-5) * g + b


def _eemb_body(eraw_ref, ew_ref, eb_ref, g_ref, b_ref, he_ref):
    h = jnp.dot(eraw_ref[...], ew_ref[...], preferred_element_type=f32)
    he_ref[...] = _ln(h + eb_ref[...], g_ref[...], b_ref[...]).astype(bf16)


def _mega_body(nraw_ref, soh_ref, mask_ref, src_ref, he_hbm,
               nw_ref, nb_ref, ws_ref, nlg_ref, nlb_ref,
               w1e_ref, w1s_ref, w1d_ref, b1_ref,
               w2_ref, b2_ref, w3_ref, b3_ref,
               g1_ref, bb1_ref, f1_ref, fb1_ref, f2_ref, fb2_ref,
               g2_ref, bb2_ref, wsk_ref, bsk_ref, gsk_ref, bsk2_ref,
               wr_ref, br_ref, p1_ref, p2_ref, p2b_ref,
               logits_ref, prj_ref,
               hv_s, a_s, bd_s, agg_s, he_buf, sem):
    h = jnp.dot(nraw_ref[...], nw_ref[...], preferred_element_type=f32)
    h = h + nb_ref[...] + jnp.dot(soh_ref[...], ws_ref[...],
                                  preferred_element_type=f32)
    hv0 = _ln(h, nlg_ref[...], nlb_ref[...]) * mask_ref[...]
    hv_s[...] = hv0
    a_s[...] = jnp.dot(hv0, w1s_ref[0], preferred_element_type=f32)
    bd_s[...] = jnp.dot(hv0, w1d_ref[0], preferred_element_type=f32)

    def he_copy(blk, slot):
        return pltpu.make_async_copy(
            he_hbm.at[pl.ds(blk * EBLK, EBLK), :], he_buf.at[slot],
            sem.at[slot])

    def layer_body(l, carry):
        w1e_b = w1e_ref[l].astype(bf16)
        b1v = b1_ref[l]
        w2v = w2_ref[l].astype(bf16)
        b2v = b2_ref[l]

        he_copy(0, 0).start()

        def edge_body(blk, c):
            slot = lax.rem(blk, 2)
            he_copy(blk, slot).wait()

            @pl.when(blk + 1 < N_EBLKS)
            def _():
                he_copy(blk + 1, lax.rem(blk + 1, 2)).start()

            base = (blk // BLKS_PER_BATCH) * L
            src_local = (src_ref[blk, 0, :] - base).astype(jnp.int16)
            oh = jnp.where(
                src_local[:, None] ==
                lax.broadcasted_iota(jnp.int16, (EBLK, L), 1),
                bf16(1.0), bf16(0.0))
            a_blk = a_s[pl.ds(base, L), :].astype(bf16)
            gath = jnp.dot(oh, a_blk, preferred_element_type=f32)
            epart = jnp.dot(he_buf[slot], w1e_b,
                            preferred_element_type=f32)
            pre = (epart + b1v + gath).reshape(NBLK, K, H) \
                + bd_s[pl.ds(blk * NBLK, NBLK), :][:, None, :]
            m = jax.nn.gelu(pre.astype(bf16)).reshape(EBLK, H)
            m2 = jnp.dot(m, w2v, preferred_element_type=f32) + b2v
            m2 = jax.nn.gelu(m2.astype(bf16)).astype(f32)
            agg_s[pl.ds(blk * NBLK, NBLK), :] = \
                jnp.sum(m2.reshape(NBLK, K, H), axis=1)
            return c

        lax.fori_loop(0, N_EBLKS, edge_body, 0)

        hv = hv_s[...]
        agg = jnp.dot(agg_s[...] * (1.0 / K), w3_ref[l],
                      preferred_element_type=f32) + b3_ref[l]
        hh = _ln(hv + agg, g1_ref[l], bb1_ref[l])
        ff = jnp.dot(jnp.maximum(
            jnp.dot(hh, f1_ref[l], preferred_element_type=f32)
            + fb1_ref[l], 0.0), f2_ref[l],
            preferred_element_type=f32) + fb2_ref[l]
        hh = _ln(hh + ff, g2_ref[l], bb2_ref[l])
        sk = jnp.maximum(jnp.dot(hh, wsk_ref[l],
                                 preferred_element_type=f32)
                         + bsk_ref[l], 0.0)
        hv_new = hv + _ln(sk, gsk_ref[l], bsk2_ref[l])
        hv_s[...] = hv_new
        ln = jnp.minimum(l + 1, NLAYERS - 1)
        a_s[...] = jnp.dot(hv_new, w1s_ref[ln], preferred_element_type=f32)
        bd_s[...] = jnp.dot(hv_new, w1d_ref[ln], preferred_element_type=f32)
        return carry

    lax.fori_loop(0, NLAYERS, layer_body, 0)

    hv = hv_s[...]
    logits_ref[...] = jnp.dot(hv, wr_ref[...],
                              preferred_element_type=f32) + br_ref[...]
    ge = jnp.mean(hv.reshape(B, L, H), axis=1)
    prj = jnp.maximum(jnp.dot(ge, p1_ref[...],
                              preferred_element_type=f32), 0.0)
    prj_ref[...] = jnp.dot(prj, p2_ref[...],
                           preferred_element_type=f32) + p2b_ref[...]


def _full(shape):
    return pl.BlockSpec(shape, lambda *_: tuple(0 for _ in shape))


def kernel(X, S, mask, params):
    p = params

    center = X[:, :, 1, :]
    diff = center[:, :, None, :] - center[:, None, :, :]
    D = jnp.sqrt(jnp.sum(diff * diff, -1) + 1e-8)
    D = D + jnp.eye(L, dtype=f32)[None] * 1e6
    negD, nbr = jax.lax.top_k(-D, K)
    d_nbr = -negD
    centers = jnp.linspace(2.0, 22.0, N_RBF)
    sigma = (22.0 - 2.0) / N_RBF
    rbf = jnp.exp(-(((d_nbr[..., None] - centers) / sigma) ** 2))
    rel = (nbr - jnp.arange(L)[None, :, None]).astype(f32)
    freq = jnp.exp(-jnp.arange(N_POS // 2, dtype=f32)
                   * (np.log(10000.0) / (N_POS // 2)))
    ang = rel[..., None] * freq
    posenc = jnp.concatenate([jnp.sin(ang), jnp.cos(ang)], -1)
    e_raw = jnp.concatenate([rbf, posenc], -1).reshape(E, EDGE_IN)

    def unit(v):
        return v / (jnp.linalg.norm(v, axis=-1, keepdims=True) + 1e-8)
    v1 = unit(X[:, :, 1] - X[:, :, 0])
    v2 = unit(X[:, :, 2] - X[:, :, 1])
    v3 = unit(jnp.roll(center, -1, axis=1) - center)
    node_raw = jnp.concatenate([v1, v2, v3], -1).reshape(N, NODE_IN)

    offs = (jnp.arange(B, dtype=jnp.int32) * L)[:, None, None]
    src = (nbr.astype(jnp.int32) + offs).reshape(N_EBLKS, 1, EBLK)
    s_oh = jax.nn.one_hot(S.reshape(N), VOCAB, dtype=f32)
    mask_col = mask.reshape(N, 1)

    row = lambda v: v.reshape(1, -1)
    layers = ['enc%d' % i for i in range(N_ENC)] + \
             ['dec%d' % i for i in range(N_DEC)]
    stk = lambda name: jnp.stack([p[pre + name] for pre in layers])
    stkr = lambda name: jnp.stack([row(p[pre + name]) for pre in layers])

    he = pl.pallas_call(
        _eemb_body,
        grid=(8,),
        in_specs=[pl.BlockSpec((E // 8, EDGE_IN), lambda i: (i, 0)),
                  _full((EDGE_IN, H)), _full((1, H)),
                  _full((1, H)), _full((1, H))],
        out_specs=pl.BlockSpec((E // 8, H), lambda i: (i, 0)),
        out_shape=jax.ShapeDtypeStruct((E, H), bf16),
        compiler_params=pltpu.CompilerParams(
            dimension_semantics=("parallel",)),
    )(e_raw, p['edge_W'], row(p['edge_b']),
      row(p['edge_ln_g']), row(p['edge_ln_b']))

    logits, prjs = pl.pallas_call(
        _mega_body,
        grid=(),
        in_specs=[_full((N, NODE_IN)), _full((N, VOCAB)), _full((N, 1)),
                  _full((N_EBLKS, 1, EBLK)),
                  pl.BlockSpec(memory_space=pl.ANY),
                  _full((NODE_IN, H)), _full((1, H)), _full((VOCAB, H)),
                  _full((1, H)), _full((1, H)),
                  _full((NLAYERS, H, H)), _full((NLAYERS, H, H)),
                  _full((NLAYERS, H, H)), _full((NLAYERS, 1, H)),
                  _full((NLAYERS, H, H)), _full((NLAYERS, 1, H)),
                  _full((NLAYERS, H, H)), _full((NLAYERS, 1, H)),
                  _full((NLAYERS, 1, H)), _full((NLAYERS, 1, H)),
                  _full((NLAYERS, H, 4 * H)), _full((NLAYERS, 1, 4 * H)),
                  _full((NLAYERS, 4 * H, H)), _full((NLAYERS, 1, H)),
                  _full((NLAYERS, 1, H)), _full((NLAYERS, 1, H)),
                  _full((NLAYERS, H, H)), _full((NLAYERS, 1, H)),
                  _full((NLAYERS, 1, H)), _full((NLAYERS, 1, H)),
                  _full((H, VOCAB)), _full((1, VOCAB)),
                  _full((H, H)), _full((H, H)), _full((1, H))],
        out_specs=[_full((N, VOCAB)), _full((B, H))],
        out_shape=[jax.ShapeDtypeStruct((N, VOCAB), f32),
                   jax.ShapeDtypeStruct((B, H), f32)],
        scratch_shapes=[pltpu.VMEM((N, H), f32), pltpu.VMEM((N, H), f32),
                        pltpu.VMEM((N, H), f32), pltpu.VMEM((N, H), f32),
                        pltpu.VMEM((2, EBLK, H), bf16),
                        pltpu.SemaphoreType.DMA((2,))],
    )(node_raw, s_oh, mask_col, src, he,
      p['node_W'], row(p['node_b']), p['W_s'],
      row(p['node_ln_g']), row(p['node_ln_b']),
      stk('_m1_W')[:, :H], stk('_m1_W')[:, H:2 * H], stk('_m1_W')[:, 2 * H:],
      stkr('_m1_b'),
      stk('_m2_W'), stkr('_m2_b'), stk('_m3_W'), stkr('_m3_b'),
      stkr('_ln1_g'), stkr('_ln1_b'),
      stk('_f1_W'), stkr('_f1_b'), stk('_f2_W'), stkr('_f2_b'),
      stkr('_ln2_g'), stkr('_ln2_b'),
      stk('_skip_W'), stkr('_skip_b'), stkr('_skln_g'), stkr('_skln_b'),
      p['readout_W'], row(p['readout_b']),
      p['proj1_W'], p['proj2_W'], row(p['proj2_b']))

    return logits, S.reshape(-1), prjs
